# baseline (speedup 1.0000x reference)
import functools

import jax
import jax.numpy as jnp
from jax.experimental import pallas as pl

NUM_EXPERTS_K = 64
TOPK_K = 8
ROW_BLOCK = 1024
SUB_ROWS = 1024


def _router_body(x_ref, w_ref, wts_ref, idx_ref, mask_ref):
    fiota = jax.lax.broadcasted_iota(
        jnp.int32, (SUB_ROWS, NUM_EXPERTS_K), 1
    ).astype(jnp.float32)
    for c in range(ROW_BLOCK // SUB_ROWS):
        rows = slice(c * SUB_ROWS, (c + 1) * SUB_ROWS)
        scores = jnp.dot(
            x_ref[rows, :], w_ref[...], preferred_element_type=jnp.float32
        )

        cur = scores
        vals = []
        idxs = []
        for _ in range(TOPK_K):
            mj = jnp.max(cur, axis=-1, keepdims=True)
            key = jnp.where(cur < mj, fiota + float(NUM_EXPERTS_K), fiota)
            ij = jnp.min(key, axis=-1, keepdims=True)
            cur = jnp.where(fiota == ij, -jnp.inf, cur)
            vals.append(mj)
            idxs.append(ij)
        top_vals = jnp.concatenate(vals, axis=1)
        top_fidx = jnp.concatenate(idxs, axis=1)

        v8 = top_vals[:, TOPK_K - 1 :]
        i8 = top_fidx[:, TOPK_K - 1 :]
        mask = jnp.where(
            (scores > v8) | ((scores == v8) & (fiota <= i8)), 1.0, 0.0
        )

        m = top_vals[:, :1]
        total = jnp.sum(jnp.exp(scores - m), axis=-1, keepdims=True)
        e = jnp.exp(top_vals - m)
        wts_ref[rows, :] = e / (jnp.sum(e, axis=1, keepdims=True) + 1e-8 * total)
        idx_ref[rows, :] = top_fidx.astype(jnp.int32)
        mask_ref[rows, :] = mask


@functools.partial(jax.jit, static_argnames=())
def kernel(x, W):
    B, S, D = x.shape
    N = B * S
    E = W.shape[1]
    x_flat = x.reshape(N, D)
    grid = (N // ROW_BLOCK,)
    wts, idx, mask = pl.pallas_call(
        _router_body,
        grid=grid,
        in_specs=[
            pl.BlockSpec((ROW_BLOCK, D), lambda i: (i, 0)),
            pl.BlockSpec((D, E), lambda i: (0, 0)),
        ],
        out_specs=[
            pl.BlockSpec((ROW_BLOCK, TOPK_K), lambda i: (i, 0)),
            pl.BlockSpec((ROW_BLOCK, TOPK_K), lambda i: (i, 0)),
            pl.BlockSpec((ROW_BLOCK, E), lambda i: (i, 0)),
        ],
        out_shape=[
            jax.ShapeDtypeStruct((N, TOPK_K), jnp.float32),
            jax.ShapeDtypeStruct((N, TOPK_K), jnp.int32),
            jax.ShapeDtypeStruct((N, E), jnp.float32),
        ],
    )(x_flat, W)
    return wts, idx, mask.reshape(B, S, E)


# speedup vs baseline: 1.4640x; 1.4640x over previous
import functools

import jax
import jax.numpy as jnp
from jax.experimental import pallas as pl

NUM_EXPERTS_K = 64
TOPK_K = 8
ROW_BLOCK = 1024


def _router_body(x_ref, w_ref, wts_ref, idx_ref, mask_ref):
    scores = jnp.dot(x_ref[...], w_ref[...], preferred_element_type=jnp.float32)
    m = jnp.max(scores, axis=-1, keepdims=True)
    e = jnp.exp(scores - m)
    probs = e / jnp.sum(e, axis=-1, keepdims=True)

    fiota = jax.lax.broadcasted_iota(jnp.int32, probs.shape, 1).astype(jnp.float32)
    cur = probs
    mask = jnp.zeros_like(probs)
    vals = []
    idxs = []
    for _ in range(TOPK_K):
        mj = jnp.max(cur, axis=-1, keepdims=True)
        is_max = cur == mj
        ij = jnp.min(
            jnp.where(is_max, fiota, float(NUM_EXPERTS_K)), axis=-1, keepdims=True
        )
        onehot = fiota == ij
        mask = jnp.where(onehot, 1.0, mask)
        cur = jnp.where(onehot, -1.0, cur)
        vals.append(mj)
        idxs.append(ij)
    top_vals = jnp.concatenate(vals, axis=1)
    top_fidx = jnp.concatenate(idxs, axis=1)
    wts_ref[...] = top_vals / (jnp.sum(top_vals, axis=1, keepdims=True) + 1e-8)
    idx_ref[...] = top_fidx.astype(jnp.int32)
    mask_ref[...] = mask


@functools.partial(jax.jit, static_argnames=())
def kernel(x, W):
    B, S, D = x.shape
    N = B * S
    E = W.shape[1]
    x_flat = x.reshape(N, D)
    grid = (N // ROW_BLOCK,)
    wts, idx, mask = pl.pallas_call(
        _router_body,
        grid=grid,
        in_specs=[
            pl.BlockSpec((ROW_BLOCK, D), lambda i: (i, 0)),
            pl.BlockSpec((D, E), lambda i: (0, 0)),
        ],
        out_specs=[
            pl.BlockSpec((ROW_BLOCK, TOPK_K), lambda i: (i, 0)),
            pl.BlockSpec((ROW_BLOCK, TOPK_K), lambda i: (i, 0)),
            pl.BlockSpec((ROW_BLOCK, E), lambda i: (i, 0)),
        ],
        out_shape=[
            jax.ShapeDtypeStruct((N, TOPK_K), jnp.float32),
            jax.ShapeDtypeStruct((N, TOPK_K), jnp.int32),
            jax.ShapeDtypeStruct((N, E), jnp.float32),
        ],
    )(x_flat, W)
    return wts, idx, mask.reshape(B, S, E)


# speedup vs baseline: 1.4646x; 1.0004x over previous
import functools

import jax
import jax.numpy as jnp
from jax.experimental import pallas as pl

NUM_EXPERTS_K = 64
TOPK_K = 8
ROW_BLOCK = 1024


def _router_body(x_ref, w_ref, wts_ref, idx_ref, mask_ref):
    scores = jnp.dot(x_ref[...], w_ref[...], preferred_element_type=jnp.float32)
    m = jnp.max(scores, axis=-1, keepdims=True)
    e = jnp.exp(scores - m)
    total = jnp.sum(e, axis=-1, keepdims=True)

    fiota = jax.lax.broadcasted_iota(jnp.int32, e.shape, 1).astype(jnp.float32)
    cur = e
    mask = jnp.zeros_like(e)
    vals = []
    idxs = []
    for _ in range(TOPK_K):
        mj = jnp.max(cur, axis=-1, keepdims=True)
        is_max = cur == mj
        ij = jnp.min(
            jnp.where(is_max, fiota, float(NUM_EXPERTS_K)), axis=-1, keepdims=True
        )
        onehot = fiota == ij
        mask = jnp.where(onehot, 1.0, mask)
        cur = jnp.where(onehot, -1.0, cur)
        vals.append(mj)
        idxs.append(ij)
    top_vals = jnp.concatenate(vals, axis=1)
    top_fidx = jnp.concatenate(idxs, axis=1)
    wts_ref[...] = top_vals / (
        jnp.sum(top_vals, axis=1, keepdims=True) + 1e-8 * total
    )
    idx_ref[...] = top_fidx.astype(jnp.int32)
    mask_ref[...] = mask


@functools.partial(jax.jit, static_argnames=())
def kernel(x, W):
    B, S, D = x.shape
    N = B * S
    E = W.shape[1]
    x_flat = x.reshape(N, D)
    grid = (N // ROW_BLOCK,)
    wts, idx, mask = pl.pallas_call(
        _router_body,
        grid=grid,
        in_specs=[
            pl.BlockSpec((ROW_BLOCK, D), lambda i: (i, 0)),
            pl.BlockSpec((D, E), lambda i: (0, 0)),
        ],
        out_specs=[
            pl.BlockSpec((ROW_BLOCK, TOPK_K), lambda i: (i, 0)),
            pl.BlockSpec((ROW_BLOCK, TOPK_K), lambda i: (i, 0)),
            pl.BlockSpec((ROW_BLOCK, E), lambda i: (i, 0)),
        ],
        out_shape=[
            jax.ShapeDtypeStruct((N, TOPK_K), jnp.float32),
            jax.ShapeDtypeStruct((N, TOPK_K), jnp.int32),
            jax.ShapeDtypeStruct((N, E), jnp.float32),
        ],
    )(x_flat, W)
    return wts, idx, mask.reshape(B, S, E)
